# baseline (speedup 1.0000x reference)
import functools

import jax
import jax.numpy as jnp
from jax import lax
from jax.experimental import pallas as pl
from jax.experimental.pallas import tpu as pltpu
from jax.experimental.pallas import tpu_sc as plsc

NUM_EMBEDDINGS = 1000000
EMBEDDING_DIM = 64
BATCH = 16384
HIST = 50

NC, NS = 2, 16
NW = NC * NS
B = BATCH * HIST
D = EMBEDDING_DIM
RPW = B // NW
TI = 128
HC = 4 * TI
NH = RPW // HC
NB = NH // 2
IDX_ROWS = RPW // TI
EO_ROWS = RPW // 2 // TI

_mesh = plsc.VectorSubcoreMesh(core_axis_name="c", subcore_axis_name="s",
                               num_cores=NC, num_subcores=NS)


@functools.partial(
    pl.kernel,
    out_type=jax.ShapeDtypeStruct((B // 2, 2 * D), jnp.float32),
    mesh=_mesh,
    compiler_params=pltpu.CompilerParams(use_tc_tiling_on_sc=False,
                                         needs_layout_passes=False),
    scratch_types=[
        pltpu.VMEM((IDX_ROWS, TI), jnp.int32),
        pltpu.VMEM((2, EO_ROWS, TI), jnp.int32),
        pltpu.VMEM((2, 2, HC // 2, D), jnp.float32),
        pltpu.SemaphoreType.DMA,
        pltpu.SemaphoreType.DMA,
        pltpu.SemaphoreType.DMA,
        pltpu.SemaphoreType.DMA,
    ],
)
def _embed_sc(idx_hbm, table_hbm, out128_hbm, idx_v, idx_eo, rows_v,
              g0, g1, o0, o1):
    wid = lax.axis_index("s") * NC + lax.axis_index("c")
    row0 = wid * RPW
    t0 = wid * IDX_ROWS

    pltpu.sync_copy(idx_hbm.at[pl.ds(pl.multiple_of(t0, 8), IDX_ROWS)], idx_v)

    lanes = lax.iota(jnp.int32, 16)

    def deint_body(m, carry):
        base = 32 * m
        for par in range(2):
            pos = base + 2 * lanes + par
            vals = plsc.load_gather(
                idx_v, [lax.shift_right_logical(pos, 7),
                        lax.bitwise_and(pos, 127)])
            idx_eo[par, lax.div(m, jnp.int32(8)),
                   pl.ds(lax.rem(m, jnp.int32(8)) * 16, 16)] = vals
        return carry

    lax.fori_loop(0, RPW // 32, deint_body, 0)

    gsems = (g0, g1)
    osems = (o0, o1)

    def fire_gathers(h, slot):
        for par in range(2):
            for j in range(2):
                pltpu.async_copy(
                    table_hbm.at[idx_eo.at[par, 2 * h + j]],
                    rows_v.at[slot, par, pl.ds(j * TI, TI)], gsems[slot])

    def drain_gathers(slot):
        for par in range(2):
            for j in range(2):
                pltpu.make_async_copy(
                    table_hbm.at[pl.ds(0, TI)],
                    rows_v.at[slot, par, pl.ds(j * TI, TI)],
                    gsems[slot]).wait()

    def fire_out(h, slot):
        off2 = pl.multiple_of((row0 + h * HC) // 2, HC // 2)
        for par in range(2):
            pltpu.async_copy(
                rows_v.at[slot, par],
                out128_hbm.at[pl.ds(off2, HC // 2), pl.ds(par * D, D)],
                osems[slot])

    def drain_out(slot):
        for par in range(2):
            pltpu.make_async_copy(
                rows_v.at[slot, par],
                out128_hbm.at[pl.ds(0, HC // 2), pl.ds(par * D, D)],
                osems[slot]).wait()

    fire_gathers(0, 0)

    def body(g, carry):
        h0 = 2 * g
        h1 = 2 * g + 1
        drain_gathers(0)
        fire_out(h0, 0)

        @pl.when(g >= 1)
        def _():
            drain_out(1)
        fire_gathers(h1, 1)

        drain_gathers(1)
        fire_out(h1, 1)

        @pl.when(g + 1 < NB)
        def _():
            drain_out(0)
            fire_gathers(h1 + 1, 0)
        return carry

    lax.fori_loop(0, NB, body, 0)
    drain_out(0)
    drain_out(1)


def kernel(input_, weight):
    idx2d = input_.reshape(B // TI, TI)
    out = _embed_sc(idx2d, weight)
    return out.reshape(BATCH, HIST, D)
